# baseline (speedup 1.0000x reference)
import jax
import jax.numpy as jnp
from jax.experimental import pallas as pl
from jax.experimental.pallas import tpu as pltpu

_B = 16
_BLK = 2048


def _total(n_ref):
    t = n_ref[0]
    for k in range(1, _B):
        t = t + n_ref[k]
    return t


def _body(n_ref, x_ref, w1_ref, b1_ref, w2_ref, b2_ref, out_ref, acc_ref):
    i = pl.program_id(0)
    nblk = pl.num_programs(0)

    @pl.when(i == 0)
    def _():
        acc_ref[...] = jnp.zeros_like(acc_ref)

    blk_start = i * _BLK
    total = _total(n_ref)

    @pl.when(blk_start < total)
    def _():
        h = jnp.maximum(
            jnp.dot(x_ref[...], w1_ref[...], preferred_element_type=jnp.float32)
            + b1_ref[...],
            0.0,
        )
        t = blk_start + jax.lax.broadcasted_iota(jnp.int32, (_BLK, 1), 0)
        seg = jnp.zeros((_BLK, 1), jnp.int32)
        e = n_ref[0]
        seg += (t >= e).astype(jnp.int32)
        for k in range(1, _B):
            e = e + n_ref[k]
            seg += (t >= e).astype(jnp.int32)
        bag = jax.lax.broadcasted_iota(jnp.int32, (_BLK, _B), 1)
        onehot = (seg == bag).astype(jnp.float32)
        acc_ref[...] += jax.lax.dot_general(
            onehot, h, (((0,), (0,)), ((), ())),
            preferred_element_type=jnp.float32,
        )

    @pl.when(i == nblk - 1)
    def _():
        out_ref[...] = (
            jnp.dot(acc_ref[...], w2_ref[...], preferred_element_type=jnp.float32)
            + b2_ref[...]
        )


def _x_map(i, n_ref):
    total = _total(n_ref)
    last = jnp.maximum(pl.cdiv(total, _BLK) - 1, 0)
    return (jnp.minimum(i, last), 0)


def kernel(x, n_instances, W1, b1, W2, b2):
    tok, d = x.shape
    h = W1.shape[1]
    out_dim = W2.shape[1]
    nblk = tok // _BLK

    grid_spec = pltpu.PrefetchScalarGridSpec(
        num_scalar_prefetch=1,
        grid=(nblk,),
        in_specs=[
            pl.BlockSpec((_BLK, d), _x_map),
            pl.BlockSpec((d, h), lambda i, n: (0, 0)),
            pl.BlockSpec((1, h), lambda i, n: (0, 0)),
            pl.BlockSpec((h, out_dim), lambda i, n: (0, 0)),
            pl.BlockSpec((1, out_dim), lambda i, n: (0, 0)),
        ],
        out_specs=pl.BlockSpec((_B, out_dim), lambda i, n: (0, 0)),
        scratch_shapes=[pltpu.VMEM((_B, h), jnp.float32)],
    )

    return pl.pallas_call(
        _body,
        grid_spec=grid_spec,
        out_shape=jax.ShapeDtypeStruct((_B, out_dim), jnp.float32),
        compiler_params=pltpu.CompilerParams(
            dimension_semantics=("arbitrary",)),
    )(n_instances, x, W1, b1.reshape(1, -1), W2, b2.reshape(1, -1))


# speedup vs baseline: 1.0375x; 1.0375x over previous
import jax
import jax.numpy as jnp
from jax.experimental import pallas as pl
from jax.experimental.pallas import tpu as pltpu

_B = 16
_BLK = 1024


def _total(n_ref):
    t = n_ref[0]
    for k in range(1, _B):
        t = t + n_ref[k]
    return t


def _body(n_ref, x_ref, w1_ref, b1_ref, w2_ref, b2_ref, out_ref, acc_ref):
    i = pl.program_id(0)
    nblk = pl.num_programs(0)

    @pl.when(i == 0)
    def _():
        acc_ref[...] = jnp.zeros_like(acc_ref)

    blk_start = i * _BLK
    total = _total(n_ref)

    @pl.when(blk_start < total)
    def _():
        h = jnp.maximum(
            jnp.dot(
                x_ref[...].astype(jnp.bfloat16),
                w1_ref[...].astype(jnp.bfloat16),
                preferred_element_type=jnp.float32,
            )
            + b1_ref[...],
            0.0,
        )
        t = blk_start + jax.lax.broadcasted_iota(jnp.int32, (_BLK, 1), 0)
        seg = jnp.zeros((_BLK, 1), jnp.int32)
        e = n_ref[0]
        seg += (t >= e).astype(jnp.int32)
        for k in range(1, _B):
            e = e + n_ref[k]
            seg += (t >= e).astype(jnp.int32)
        bag = jax.lax.broadcasted_iota(jnp.int32, (_BLK, _B), 1)
        onehot = (seg == bag).astype(jnp.float32)
        acc_ref[...] += jax.lax.dot_general(
            onehot, h, (((0,), (0,)), ((), ())),
            preferred_element_type=jnp.float32,
        )

    @pl.when(i == nblk - 1)
    def _():
        out_ref[...] = (
            jnp.dot(acc_ref[...], w2_ref[...], preferred_element_type=jnp.float32)
            + b2_ref[...]
        )


def _x_map(i, n_ref):
    total = _total(n_ref)
    last = jnp.maximum(pl.cdiv(total, _BLK) - 1, 0)
    return (jnp.minimum(i, last), 0)


def kernel(x, n_instances, W1, b1, W2, b2):
    tok, d = x.shape
    h = W1.shape[1]
    out_dim = W2.shape[1]
    nblk = tok // _BLK

    grid_spec = pltpu.PrefetchScalarGridSpec(
        num_scalar_prefetch=1,
        grid=(nblk,),
        in_specs=[
            pl.BlockSpec((_BLK, d), _x_map),
            pl.BlockSpec((d, h), lambda i, n: (0, 0)),
            pl.BlockSpec((1, h), lambda i, n: (0, 0)),
            pl.BlockSpec((h, out_dim), lambda i, n: (0, 0)),
            pl.BlockSpec((1, out_dim), lambda i, n: (0, 0)),
        ],
        out_specs=pl.BlockSpec((_B, out_dim), lambda i, n: (0, 0)),
        scratch_shapes=[pltpu.VMEM((_B, h), jnp.float32)],
    )

    return pl.pallas_call(
        _body,
        grid_spec=grid_spec,
        out_shape=jax.ShapeDtypeStruct((_B, out_dim), jnp.float32),
        compiler_params=pltpu.CompilerParams(
            dimension_semantics=("arbitrary",)),
    )(n_instances, x, W1, b1.reshape(1, -1), W2, b2.reshape(1, -1))


# speedup vs baseline: 1.3877x; 1.3375x over previous
import jax
import jax.numpy as jnp
from jax.experimental import pallas as pl
from jax.experimental.pallas import tpu as pltpu

_B = 16
_BLK = 1024


def _total(n_ref):
    t = n_ref[0]
    for k in range(1, _B):
        t = t + n_ref[k]
    return t


def _body(n_ref, x_ref, w1_ref, b1_ref, w2_ref, b2_ref, out_ref, acc_ref):
    i = pl.program_id(0)
    nblk = pl.num_programs(0)

    @pl.when(i == 0)
    def _():
        acc_ref[...] = jnp.zeros_like(acc_ref)

    blk_start = i * _BLK
    total = _total(n_ref)

    @pl.when(blk_start < total)
    def _():
        h = jnp.maximum(
            jnp.dot(
                x_ref[...].astype(jnp.bfloat16),
                w1_ref[...].astype(jnp.bfloat16),
                preferred_element_type=jnp.float32,
            )
            + b1_ref[...],
            0.0,
        )
        t_row = blk_start + jax.lax.broadcasted_iota(jnp.int32, (_B, _BLK), 1)
        row = jax.lax.broadcasted_iota(jnp.int32, (_B, 1), 0)
        starts = jnp.zeros((_B, 1), jnp.int32)
        ends = jnp.zeros((_B, 1), jnp.int32)
        e = n_ref[0]
        ends = jnp.where(row == 0, e, ends)
        for k in range(1, _B):
            s = e
            e = e + n_ref[k]
            starts = jnp.where(row == k, s, starts)
            ends = jnp.where(row == k, e, ends)
        onehot = ((t_row >= starts) & (t_row < ends)).astype(jnp.float32)
        acc_ref[...] += jax.lax.dot_general(
            onehot, h, (((1,), (0,)), ((), ())),
            preferred_element_type=jnp.float32,
        )

    @pl.when(i == nblk - 1)
    def _():
        out_ref[...] = (
            jnp.dot(acc_ref[...], w2_ref[...], preferred_element_type=jnp.float32)
            + b2_ref[...]
        )


def _x_map(i, n_ref):
    total = _total(n_ref)
    last = jnp.maximum(pl.cdiv(total, _BLK) - 1, 0)
    return (jnp.minimum(i, last), 0)


def kernel(x, n_instances, W1, b1, W2, b2):
    tok, d = x.shape
    h = W1.shape[1]
    out_dim = W2.shape[1]
    nblk = tok // _BLK

    grid_spec = pltpu.PrefetchScalarGridSpec(
        num_scalar_prefetch=1,
        grid=(nblk,),
        in_specs=[
            pl.BlockSpec((_BLK, d), _x_map),
            pl.BlockSpec((d, h), lambda i, n: (0, 0)),
            pl.BlockSpec((1, h), lambda i, n: (0, 0)),
            pl.BlockSpec((h, out_dim), lambda i, n: (0, 0)),
            pl.BlockSpec((1, out_dim), lambda i, n: (0, 0)),
        ],
        out_specs=pl.BlockSpec((_B, out_dim), lambda i, n: (0, 0)),
        scratch_shapes=[pltpu.VMEM((_B, h), jnp.float32)],
    )

    return pl.pallas_call(
        _body,
        grid_spec=grid_spec,
        out_shape=jax.ShapeDtypeStruct((_B, out_dim), jnp.float32),
        compiler_params=pltpu.CompilerParams(
            dimension_semantics=("arbitrary",)),
    )(n_instances, x, W1, b1.reshape(1, -1), W2, b2.reshape(1, -1))
